# baseline (speedup 1.0000x reference)
import functools

import jax
import jax.numpy as jnp
from jax import lax
from jax.experimental import pallas as pl
from jax.experimental.pallas import tpu as pltpu
from jax.experimental.pallas import tpu_sc as plsc

N = 10000
E = 320000
D = 128
DE = 16
H = 128

NC = 2
NS = 16
NW = NC * NS
EPW = E // NW
RG = 400
RS = 200
LANES = 16


@functools.cache
def _sc_mesh():
    return plsc.VectorSubcoreMesh(core_axis_name="c", subcore_axis_name="s",
                                  num_cores=NC, num_subcores=NS)


def _pq_body(x, w1a, w1b, p, q):
    f32 = jnp.float32
    xb = x[...].astype(jnp.bfloat16)
    p[...] = jnp.dot(xb, w1a[...], preferred_element_type=f32)
    q[...] = jnp.dot(xb, w1b[...], preferred_element_type=f32)


def _pq(x, w1a, w1b):
    BN = 1000
    return pl.pallas_call(
        _pq_body,
        grid=(N // BN,),
        in_specs=[pl.BlockSpec((BN, D), lambda i: (i, 0)),
                  pl.BlockSpec((D, H), lambda i: (0, 0)),
                  pl.BlockSpec((D, H), lambda i: (0, 0))],
        out_specs=[pl.BlockSpec((BN, H), lambda i: (i, 0)),
                   pl.BlockSpec((BN, H), lambda i: (i, 0))],
        out_shape=[jax.ShapeDtypeStruct((N, H), jnp.float32),
                   jax.ShapeDtypeStruct((N, H), jnp.float32)],
    )(x, w1a, w1b)


@functools.cache
def _gather_kernel():
    @functools.partial(
        pl.kernel,
        out_type=jax.ShapeDtypeStruct((E, D), jnp.float32),
        mesh=_sc_mesh(),
        scratch_types=[
            pltpu.VMEM((RG,), jnp.int32),
            pltpu.VMEM((RG,), jnp.int32),
            pltpu.VMEM((RG, D), jnp.float32),
            pltpu.VMEM((RG, D), jnp.float32),
            pltpu.SemaphoreType.DMA,
            pltpu.SemaphoreType.DMA,
        ],
    )
    def _gather(p_hbm, q_hbm, dst_hbm, src_hbm, g_hbm,
                idxd_v, idxs_v, bufp_v, bufq_v, semp, semq):
        wid = lax.axis_index("s") * NC + lax.axis_index("c")
        base = wid * EPW

        def body(i, carry):
            off = base + i * RG
            pltpu.sync_copy(dst_hbm.at[pl.ds(off, RG)], idxd_v)
            pltpu.sync_copy(src_hbm.at[pl.ds(off, RG)], idxs_v)
            cp = pltpu.async_copy(p_hbm.at[idxd_v], bufp_v, semp)
            cq = pltpu.async_copy(q_hbm.at[idxs_v], bufq_v, semq)
            cp.wait()
            cq.wait()

            def row(j, carry2):
                for k in range(D // LANES):
                    s = pl.ds(k * LANES, LANES)
                    bufp_v[j, s] = bufp_v[j, s] + bufq_v[j, s]
                return carry2

            lax.fori_loop(0, RG, row, 0)
            pltpu.sync_copy(bufp_v, g_hbm.at[pl.ds(off, RG)])
            return carry

        lax.fori_loop(0, EPW // RG, body, 0)

    return _gather


BE = 512


def _mlp_body(g, eat, lat, uc, w1c, b1, w2, b2, w3, b3, out):
    f32 = jnp.float32
    bf = jnp.bfloat16
    a = lax.dot_general(eat[...], w1c[...], (((0,), (0,)), ((), ())),
                        preferred_element_type=f32)
    h = g[...] + a + b1[...]
    h = h * jax.nn.sigmoid(h)
    h = jnp.dot(h.astype(bf), w2[...], preferred_element_type=f32) + b2[...]
    h = h * jax.nn.sigmoid(h)
    m = jnp.dot(h.astype(bf), w3[...], preferred_element_type=f32) + b3[...]
    mu = jnp.mean(m, axis=-1, keepdims=True)
    var = jnp.mean((m - mu) * (m - mu), axis=-1, keepdims=True)
    msg = (m - mu) * lax.rsqrt(var + 1e-5)
    u = uc[...].reshape(-1, 1)
    co = lax.rsqrt(u * u + 1.0)
    out[...] = co * lat[...] + (u * co) * msg


def _mlp(g, eat, lat, uc, w1c, b1, w2, b2, w3, b3):
    grid = (E // BE,)
    blk = lambda r, c: pl.BlockSpec((r, c), lambda i: (i, 0))
    whole = lambda r, c: pl.BlockSpec((r, c), lambda i: (0, 0))
    return pl.pallas_call(
        _mlp_body,
        grid=grid,
        in_specs=[
            blk(BE, D),
            pl.BlockSpec((DE, BE), lambda i: (0, i)),
            blk(BE, D),
            pl.BlockSpec((BE,), lambda i: (i,)),
            whole(DE, H), whole(1, H),
            whole(H, H), whole(1, H), whole(H, D), whole(1, D),
        ],
        out_specs=blk(BE, D),
        out_shape=jax.ShapeDtypeStruct((E, D), jnp.float32),
    )(g, eat, lat, uc, w1c, b1, w2, b2, w3, b3)


@functools.cache
def _scatter_kernel():
    @functools.partial(
        pl.kernel,
        out_type=jax.ShapeDtypeStruct((NC * N, D), jnp.float32),
        mesh=_sc_mesh(),
        scratch_types=[
            pltpu.VMEM((RS,), jnp.int32),
            pltpu.VMEM((RS, D), jnp.float32),
            pltpu.VMEM_SHARED((N, D), jnp.float32),
            pltpu.SemaphoreType.DMA,
        ],
    )
    def _scatter(lat_hbm, dst_hbm, zeros_hbm, out_hbm, idx_v, buf_v, acc_sh, sem):
        cid = lax.axis_index("c")
        sid = lax.axis_index("s")
        wid = sid * NC + cid

        @pl.when(sid < 15)
        def _():
            pltpu.sync_copy(zeros_hbm.at[pl.ds(sid * 640, 640)],
                            acc_sh.at[pl.ds(sid * 640, 640)])

        @pl.when(sid == 15)
        def _():
            pltpu.sync_copy(zeros_hbm.at[pl.ds(9600, 400)],
                            acc_sh.at[pl.ds(9600, 400)])

        plsc.subcore_barrier()

        def body(i, carry):
            off = wid * EPW + i * RS
            pltpu.sync_copy(dst_hbm.at[pl.ds(off, RS)], idx_v)
            pltpu.sync_copy(lat_hbm.at[pl.ds(off, RS)], buf_v)
            pltpu.sync_copy(buf_v, acc_sh.at[idx_v], add=True)
            return carry

        lax.fori_loop(0, EPW // RS, body, 0)
        plsc.subcore_barrier()

        @pl.when(sid < 15)
        def _():
            pltpu.sync_copy(acc_sh.at[pl.ds(sid * 640, 640)],
                            out_hbm.at[pl.ds(cid * N + sid * 640, 640)])

        @pl.when(sid == 15)
        def _():
            pltpu.sync_copy(acc_sh.at[pl.ds(9600, 400)],
                            out_hbm.at[pl.ds(cid * N + 9600, 400)])

    return _scatter


def _add_body(a, b, out):
    out[...] = a[...] + b[...]


def _final_add(p0, p1):
    BN = 200
    return pl.pallas_call(
        _add_body,
        grid=(N // BN,),
        in_specs=[pl.BlockSpec((BN, D), lambda i: (i, 0)),
                  pl.BlockSpec((BN, D), lambda i: (i, 0))],
        out_specs=pl.BlockSpec((BN, D), lambda i: (i, 0)),
        out_shape=jax.ShapeDtypeStruct((N, D), jnp.float32),
    )(p0, p1)


def kernel(x, edge_index, edge_attr, latents, update_coeff, W1, b1, W2, b2, W3, b3):
    bf = jnp.bfloat16
    src = edge_index[0]
    dst = edge_index[1]
    w1a = W1[:D].astype(bf)
    w1b = W1[D:2 * D].astype(bf)
    w1c = W1[2 * D:]
    p, q = _pq(x, w1a, w1b)
    g = _gather_kernel()(p, q, dst, src)
    new_lat = _mlp(g, edge_attr.T, latents, update_coeff.reshape(E),
                   w1c, b1.reshape(1, H),
                   W2.astype(bf), b2.reshape(1, H), W3.astype(bf),
                   b3.reshape(1, D))
    zeros = jnp.zeros((N, D), jnp.float32)
    partials = _scatter_kernel()(new_lat, dst, zeros)
    return _final_add(partials[:N], partials[N:])


# speedup vs baseline: 1.2100x; 1.2100x over previous
import functools

import jax
import jax.numpy as jnp
from jax import lax
from jax.experimental import pallas as pl
from jax.experimental.pallas import tpu as pltpu
from jax.experimental.pallas import tpu_sc as plsc

N = 10000
E = 320000
D = 128
DE = 16
H = 128

NC = 2
NS = 16
NW = NC * NS
EPW = E // NW
RG = 400
RS = 200
LANES = 16


@functools.cache
def _sc_mesh():
    return plsc.VectorSubcoreMesh(core_axis_name="c", subcore_axis_name="s",
                                  num_cores=NC, num_subcores=NS)


def _pq_body(x, w1a, w1b, p, q):
    f32 = jnp.float32
    xb = x[...].astype(jnp.bfloat16)
    p[...] = jnp.dot(xb, w1a[...], preferred_element_type=f32)
    q[...] = jnp.dot(xb, w1b[...], preferred_element_type=f32)


def _pq(x, w1a, w1b):
    BN = 1000
    return pl.pallas_call(
        _pq_body,
        grid=(N // BN,),
        in_specs=[pl.BlockSpec((BN, D), lambda i: (i, 0)),
                  pl.BlockSpec((D, H), lambda i: (0, 0)),
                  pl.BlockSpec((D, H), lambda i: (0, 0))],
        out_specs=[pl.BlockSpec((BN, H), lambda i: (i, 0)),
                   pl.BlockSpec((BN, H), lambda i: (i, 0))],
        out_shape=[jax.ShapeDtypeStruct((N, H), jnp.float32),
                   jax.ShapeDtypeStruct((N, H), jnp.float32)],
    )(x, w1a, w1b)


@functools.cache
def _gather_kernel():
    @functools.partial(
        pl.kernel,
        out_type=jax.ShapeDtypeStruct((E, D), jnp.float32),
        mesh=_sc_mesh(),
        scratch_types=[
            pltpu.VMEM((RG,), jnp.int32),
            pltpu.VMEM((RG,), jnp.int32),
            pltpu.VMEM((RG, D), jnp.float32),
            pltpu.VMEM((RG, D), jnp.float32),
            pltpu.SemaphoreType.DMA,
            pltpu.SemaphoreType.DMA,
        ],
    )
    def _gather(p_hbm, q_hbm, dst_hbm, src_hbm, g_hbm,
                idxd_v, idxs_v, bufp_v, bufq_v, semp, semq):
        wid = lax.axis_index("s") * NC + lax.axis_index("c")
        base = wid * EPW

        def body(i, carry):
            off = base + i * RG
            pltpu.sync_copy(dst_hbm.at[pl.ds(off, RG)], idxd_v)
            pltpu.sync_copy(src_hbm.at[pl.ds(off, RG)], idxs_v)
            cp = pltpu.async_copy(p_hbm.at[idxd_v], bufp_v, semp)
            cq = pltpu.async_copy(q_hbm.at[idxs_v], bufq_v, semq)
            cp.wait()
            cq.wait()

            def row(j, carry2):
                for k in range(D // LANES):
                    s = pl.ds(k * LANES, LANES)
                    bufp_v[j, s] = bufp_v[j, s] + bufq_v[j, s]
                return carry2

            lax.fori_loop(0, RG, row, 0)
            pltpu.sync_copy(bufp_v, g_hbm.at[pl.ds(off, RG)])
            return carry

        lax.fori_loop(0, EPW // RG, body, 0)

    return _gather


BE = 1280


def _mlp_body(g, eat, lat, uc, w1c, b1, w2, b2, w3, b3, out):
    f32 = jnp.float32
    bf = jnp.bfloat16
    a = lax.dot_general(eat[...], w1c[...], (((0,), (0,)), ((), ())),
                        preferred_element_type=f32)
    h = g[...] + a + b1[...]
    h = h * jax.nn.sigmoid(h)
    h = jnp.dot(h.astype(bf), w2[...], preferred_element_type=f32) + b2[...]
    h = h * jax.nn.sigmoid(h)
    m = jnp.dot(h.astype(bf), w3[...], preferred_element_type=f32) + b3[...]
    mu = jnp.mean(m, axis=-1, keepdims=True)
    var = jnp.mean((m - mu) * (m - mu), axis=-1, keepdims=True)
    msg = (m - mu) * lax.rsqrt(var + 1e-5)
    u = uc[...]
    co = lax.rsqrt(u * u + 1.0)
    out[...] = co * lat[...] + (u * co) * msg


def _mlp(g, eat, lat, uc, w1c, b1, w2, b2, w3, b3):
    grid = (E // BE,)
    blk = lambda r, c: pl.BlockSpec((r, c), lambda i: (i, 0))
    whole = lambda r, c: pl.BlockSpec((r, c), lambda i: (0, 0))
    return pl.pallas_call(
        _mlp_body,
        grid=grid,
        in_specs=[
            blk(BE, D),
            pl.BlockSpec((DE, BE), lambda i: (0, i)),
            blk(BE, D),
            blk(BE, 1),
            whole(DE, H), whole(1, H),
            whole(H, H), whole(1, H), whole(H, D), whole(1, D),
        ],
        out_specs=blk(BE, D),
        out_shape=jax.ShapeDtypeStruct((E, D), jnp.float32),
    )(g, eat, lat, uc, w1c, b1, w2, b2, w3, b3)


@functools.cache
def _scatter_kernel():
    @functools.partial(
        pl.kernel,
        out_type=jax.ShapeDtypeStruct((NC * N, D), jnp.float32),
        mesh=_sc_mesh(),
        scratch_types=[
            pltpu.VMEM((RS,), jnp.int32),
            pltpu.VMEM((RS, D), jnp.float32),
            pltpu.VMEM_SHARED((N, D), jnp.float32),
            pltpu.SemaphoreType.DMA,
        ],
    )
    def _scatter(lat_hbm, dst_hbm, zeros_hbm, out_hbm, idx_v, buf_v, acc_sh, sem):
        cid = lax.axis_index("c")
        sid = lax.axis_index("s")
        wid = sid * NC + cid

        @pl.when(sid < 15)
        def _():
            pltpu.sync_copy(zeros_hbm.at[pl.ds(sid * 640, 640)],
                            acc_sh.at[pl.ds(sid * 640, 640)])

        @pl.when(sid == 15)
        def _():
            pltpu.sync_copy(zeros_hbm.at[pl.ds(9600, 400)],
                            acc_sh.at[pl.ds(9600, 400)])

        plsc.subcore_barrier()

        def body(i, carry):
            off = wid * EPW + i * RS
            pltpu.sync_copy(dst_hbm.at[pl.ds(off, RS)], idx_v)
            pltpu.sync_copy(lat_hbm.at[pl.ds(off, RS)], buf_v)
            pltpu.sync_copy(buf_v, acc_sh.at[idx_v], add=True)
            return carry

        lax.fori_loop(0, EPW // RS, body, 0)
        plsc.subcore_barrier()

        @pl.when(sid < 15)
        def _():
            pltpu.sync_copy(acc_sh.at[pl.ds(sid * 640, 640)],
                            out_hbm.at[pl.ds(cid * N + sid * 640, 640)])

        @pl.when(sid == 15)
        def _():
            pltpu.sync_copy(acc_sh.at[pl.ds(9600, 400)],
                            out_hbm.at[pl.ds(cid * N + 9600, 400)])

    return _scatter


def _add_body(a, b, out):
    out[...] = a[...] + b[...]


def _final_add(p0, p1):
    BN = 200
    return pl.pallas_call(
        _add_body,
        grid=(N // BN,),
        in_specs=[pl.BlockSpec((BN, D), lambda i: (i, 0)),
                  pl.BlockSpec((BN, D), lambda i: (i, 0))],
        out_specs=pl.BlockSpec((BN, D), lambda i: (i, 0)),
        out_shape=jax.ShapeDtypeStruct((N, D), jnp.float32),
    )(p0, p1)


def kernel(x, edge_index, edge_attr, latents, update_coeff, W1, b1, W2, b2, W3, b3):
    bf = jnp.bfloat16
    src = edge_index[0]
    dst = edge_index[1]
    w1a = W1[:D].astype(bf)
    w1b = W1[D:2 * D].astype(bf)
    w1c = W1[2 * D:]
    p, q = _pq(x, w1a, w1b)
    g = _gather_kernel()(p, q, dst, src)
    new_lat = _mlp(g, edge_attr.T, latents, update_coeff,
                   w1c, b1.reshape(1, H),
                   W2.astype(bf), b2.reshape(1, H), W3.astype(bf),
                   b3.reshape(1, D))
    zeros = jnp.zeros((N, D), jnp.float32)
    partials = _scatter_kernel()(new_lat, dst, zeros)
    return _final_add(partials[:N], partials[N:])


# speedup vs baseline: 1.3834x; 1.1432x over previous
import functools

import jax
import jax.numpy as jnp
from jax import lax
from jax.experimental import pallas as pl
from jax.experimental.pallas import tpu as pltpu
from jax.experimental.pallas import tpu_sc as plsc

N = 10000
E = 320000
D = 128
DE = 16
H = 128

K = 5
CE = E // K
NC = 2
NS = 16
NW = NC * NS
EPW = CE // NW
RG = 400
RS = 200
LANES = 16
ZR = 80


@functools.cache
def _sc_mesh():
    return plsc.VectorSubcoreMesh(core_axis_name="c", subcore_axis_name="s",
                                  num_cores=NC, num_subcores=NS)


def _pq_body(x, w1a, w1b, p, q):
    f32 = jnp.float32
    xb = x[...].astype(jnp.bfloat16)
    p[...] = jnp.dot(xb, w1a[...], preferred_element_type=f32)
    q[...] = jnp.dot(xb, w1b[...], preferred_element_type=f32)


def _pq(x, w1a, w1b):
    BN = 1000
    return pl.pallas_call(
        _pq_body,
        grid=(N // BN,),
        in_specs=[pl.BlockSpec((BN, D), lambda i: (i, 0)),
                  pl.BlockSpec((D, H), lambda i: (0, 0)),
                  pl.BlockSpec((D, H), lambda i: (0, 0))],
        out_specs=[pl.BlockSpec((BN, H), lambda i: (i, 0)),
                   pl.BlockSpec((BN, H), lambda i: (i, 0))],
        out_shape=[jax.ShapeDtypeStruct((N, H), jnp.float32),
                   jax.ShapeDtypeStruct((N, H), jnp.float32)],
    )(x, w1a, w1b)


@functools.cache
def _gather_kernel():
    @functools.partial(
        pl.kernel,
        out_type=jax.ShapeDtypeStruct((CE, D), jnp.float32),
        mesh=_sc_mesh(),
        scratch_types=[
            pltpu.VMEM((RG,), jnp.int32),
            pltpu.VMEM((RG,), jnp.int32),
            pltpu.VMEM((RG, D), jnp.float32),
            pltpu.VMEM((RG, D), jnp.float32),
            pltpu.SemaphoreType.DMA,
            pltpu.SemaphoreType.DMA,
        ],
    )
    def _gather(p_hbm, q_hbm, dst_hbm, src_hbm, g_hbm,
                idxd_v, idxs_v, bufp_v, bufq_v, semp, semq):
        wid = lax.axis_index("s") * NC + lax.axis_index("c")
        base = wid * EPW

        def body(i, carry):
            off = base + i * RG
            pltpu.sync_copy(dst_hbm.at[pl.ds(off, RG)], idxd_v)
            pltpu.sync_copy(src_hbm.at[pl.ds(off, RG)], idxs_v)
            cp = pltpu.async_copy(p_hbm.at[idxd_v], bufp_v, semp)
            cq = pltpu.async_copy(q_hbm.at[idxs_v], bufq_v, semq)
            cp.wait()
            cq.wait()

            def row(j, carry2):
                for k in range(D // LANES):
                    s = pl.ds(k * LANES, LANES)
                    bufp_v[j, s] = bufp_v[j, s] + bufq_v[j, s]
                return carry2

            lax.fori_loop(0, RG, row, 0)
            pltpu.sync_copy(bufp_v, g_hbm.at[pl.ds(off, RG)])
            return carry

        lax.fori_loop(0, EPW // RG, body, 0)

    return _gather


BE = 1280


def _mlp_body(g, eat, lat, uc, w1c, b1, w2, b2, w3, b3, out):
    f32 = jnp.float32
    bf = jnp.bfloat16
    a = lax.dot_general(eat[...], w1c[...], (((0,), (0,)), ((), ())),
                        preferred_element_type=f32)
    h = g[...] + a + b1[...]
    h = h * jax.nn.sigmoid(h)
    h = jnp.dot(h.astype(bf), w2[...], preferred_element_type=f32) + b2[...]
    h = h * jax.nn.sigmoid(h)
    m = jnp.dot(h.astype(bf), w3[...], preferred_element_type=f32) + b3[...]
    mu = jnp.mean(m, axis=-1, keepdims=True)
    var = jnp.mean((m - mu) * (m - mu), axis=-1, keepdims=True)
    msg = (m - mu) * lax.rsqrt(var + 1e-5)
    u = uc[...]
    co = lax.rsqrt(u * u + 1.0)
    out[...] = co * lat[...] + (u * co) * msg


def _mlp(g, eat, lat, uc, w1c, b1, w2, b2, w3, b3):
    grid = (CE // BE,)
    blk = lambda r, c: pl.BlockSpec((r, c), lambda i: (i, 0))
    whole = lambda r, c: pl.BlockSpec((r, c), lambda i: (0, 0))
    return pl.pallas_call(
        _mlp_body,
        grid=grid,
        in_specs=[
            blk(BE, D),
            pl.BlockSpec((DE, BE), lambda i: (0, i)),
            blk(BE, D),
            blk(BE, 1),
            whole(DE, H), whole(1, H),
            whole(H, H), whole(1, H), whole(H, D), whole(1, D),
        ],
        out_specs=blk(BE, D),
        out_shape=jax.ShapeDtypeStruct((CE, D), jnp.float32),
    )(g, eat, lat, uc, w1c, b1, w2, b2, w3, b3)


@functools.cache
def _scatter_kernel():
    @functools.partial(
        pl.kernel,
        out_type=jax.ShapeDtypeStruct((NC * N, D), jnp.float32),
        mesh=_sc_mesh(),
        scratch_types=[
            pltpu.VMEM((RS,), jnp.int32),
            pltpu.VMEM((RS, D), jnp.float32),
            pltpu.VMEM_SHARED((N, D), jnp.float32),
            pltpu.SemaphoreType.DMA,
        ],
    )
    def _scatter(lat_hbm, dst_hbm, out_hbm, idx_v, buf_v, acc_sh, sem):
        cid = lax.axis_index("c")
        sid = lax.axis_index("s")
        wid = sid * NC + cid

        zero = jnp.zeros((LANES,), jnp.float32)

        def zrow(j, c2):
            for k in range(D // LANES):
                buf_v[j, pl.ds(k * LANES, LANES)] = zero
            return c2

        lax.fori_loop(0, ZR, zrow, 0)
        for k in range(640 // ZR):
            @pl.when((sid < 15) | (k < 400 // ZR))
            def _():
                pltpu.sync_copy(buf_v.at[pl.ds(0, ZR)],
                                acc_sh.at[pl.ds(sid * 640 + k * ZR, ZR)])

        plsc.subcore_barrier()

        def body(i, carry):
            off = wid * EPW + i * RS
            pltpu.sync_copy(dst_hbm.at[pl.ds(off, RS)], idx_v)
            pltpu.sync_copy(lat_hbm.at[pl.ds(off, RS)], buf_v)
            pltpu.sync_copy(buf_v, acc_sh.at[idx_v], add=True)
            return carry

        lax.fori_loop(0, EPW // RS, body, 0)
        plsc.subcore_barrier()

        @pl.when(sid < 15)
        def _():
            pltpu.sync_copy(acc_sh.at[pl.ds(sid * 640, 640)],
                            out_hbm.at[pl.ds(cid * N + sid * 640, 640)])

        @pl.when(sid == 15)
        def _():
            pltpu.sync_copy(acc_sh.at[pl.ds(9600, 400)],
                            out_hbm.at[pl.ds(cid * N + 9600, 400)])

    return _scatter


def _add_body(*refs):
    out = refs[-1]
    acc = refs[0][...]
    for r in refs[1:-1]:
        acc = acc + r[...]
    out[...] = acc


def _final_add(parts):
    BN = 1000
    return pl.pallas_call(
        _add_body,
        grid=(N // BN,),
        in_specs=[pl.BlockSpec((BN, D), lambda i: (i, 0)) for _ in parts],
        out_specs=pl.BlockSpec((BN, D), lambda i: (i, 0)),
        out_shape=jax.ShapeDtypeStruct((N, D), jnp.float32),
    )(*parts)


def kernel(x, edge_index, edge_attr, latents, update_coeff, W1, b1, W2, b2, W3, b3):
    bf = jnp.bfloat16
    src = edge_index[0]
    dst = edge_index[1]
    w1a = W1[:D].astype(bf)
    w1b = W1[D:2 * D].astype(bf)
    w1c = W1[2 * D:]
    eat = edge_attr.T
    p, q = _pq(x, w1a, w1b)

    gather = _gather_kernel()
    scatter = _scatter_kernel()
    parts = []
    for c in range(K):
        lo, hi = c * CE, (c + 1) * CE
        g = gather(p, q, dst[lo:hi], src[lo:hi])
        new_lat = _mlp(g, eat[:, lo:hi], latents[lo:hi], update_coeff[lo:hi],
                       w1c, b1.reshape(1, H),
                       W2.astype(bf), b2.reshape(1, H), W3.astype(bf),
                       b3.reshape(1, D))
        pc = scatter(new_lat, dst[lo:hi])
        parts.append(pc[:N])
        parts.append(pc[N:])
    return _final_add(parts)


# speedup vs baseline: 1.6127x; 1.1658x over previous
import functools

import jax
import jax.numpy as jnp
from jax import lax
from jax.experimental import pallas as pl
from jax.experimental.pallas import tpu as pltpu
from jax.experimental.pallas import tpu_sc as plsc

N = 10000
E = 320000
D = 128
DE = 16
H = 128

K = 5
CE = E // K
NC = 2
NS = 16
NW = NC * NS
EPW = CE // NW
RG = 400
RS = 200
LANES = 16
ZR = 80


@functools.cache
def _sc_mesh():
    return plsc.VectorSubcoreMesh(core_axis_name="c", subcore_axis_name="s",
                                  num_cores=NC, num_subcores=NS)


def _pq_body(x, w1a, w1b, p, q):
    f32 = jnp.float32
    xb = x[...].astype(jnp.bfloat16)
    p[...] = jnp.dot(xb, w1a[...], preferred_element_type=f32)
    q[...] = jnp.dot(xb, w1b[...], preferred_element_type=f32)


def _pq(x, w1a, w1b):
    BN = 1000
    return pl.pallas_call(
        _pq_body,
        grid=(N // BN,),
        in_specs=[pl.BlockSpec((BN, D), lambda i: (i, 0)),
                  pl.BlockSpec((D, H), lambda i: (0, 0)),
                  pl.BlockSpec((D, H), lambda i: (0, 0))],
        out_specs=[pl.BlockSpec((BN, H), lambda i: (i, 0)),
                   pl.BlockSpec((BN, H), lambda i: (i, 0))],
        out_shape=[jax.ShapeDtypeStruct((N, H), jnp.float32),
                   jax.ShapeDtypeStruct((N, H), jnp.float32)],
    )(x, w1a, w1b)


@functools.cache
def _gather_kernel(chunk):
    @functools.partial(
        pl.kernel,
        out_type=jax.ShapeDtypeStruct((CE, D), jnp.float32),
        mesh=_sc_mesh(),
        scratch_types=[
            pltpu.VMEM((RG,), jnp.int32),
            pltpu.VMEM((RG,), jnp.int32),
            pltpu.VMEM((RG, D), jnp.float32),
            pltpu.VMEM((RG, D), jnp.float32),
            pltpu.SemaphoreType.DMA,
            pltpu.SemaphoreType.DMA,
        ],
    )
    def _gather(p_hbm, q_hbm, dst_hbm, src_hbm, g_hbm,
                idxd_v, idxs_v, bufp_v, bufq_v, semp, semq):
        wid = lax.axis_index("s") * NC + lax.axis_index("c")
        ebase = chunk * CE + wid * EPW

        def body(i, carry):
            off = ebase + i * RG
            pltpu.sync_copy(dst_hbm.at[pl.ds(off, RG)], idxd_v)
            pltpu.sync_copy(src_hbm.at[pl.ds(off, RG)], idxs_v)
            cp = pltpu.async_copy(p_hbm.at[idxd_v], bufp_v, semp)
            cq = pltpu.async_copy(q_hbm.at[idxs_v], bufq_v, semq)
            cp.wait()
            cq.wait()

            def row(j, carry2):
                for k in range(D // LANES):
                    s = pl.ds(k * LANES, LANES)
                    bufp_v[j, s] = bufp_v[j, s] + bufq_v[j, s]
                return carry2

            lax.fori_loop(0, RG, row, 0)
            pltpu.sync_copy(bufp_v, g_hbm.at[pl.ds(off - chunk * CE, RG)])
            return carry

        lax.fori_loop(0, EPW // RG, body, 0)

    return _gather


BE = 1280


def _mlp_body(g, eat, lat, uc, w1c, b1, w2, b2, w3, b3, out):
    f32 = jnp.float32
    bf = jnp.bfloat16
    a = lax.dot_general(eat[...], w1c[...], (((0,), (0,)), ((), ())),
                        preferred_element_type=f32)
    h = g[...] + a + b1[...]
    h = h * jax.nn.sigmoid(h)
    h = jnp.dot(h.astype(bf), w2[...], preferred_element_type=f32) + b2[...]
    h = h * jax.nn.sigmoid(h)
    m = jnp.dot(h.astype(bf), w3[...], preferred_element_type=f32) + b3[...]
    mu = jnp.mean(m, axis=-1, keepdims=True)
    var = jnp.mean((m - mu) * (m - mu), axis=-1, keepdims=True)
    msg = (m - mu) * lax.rsqrt(var + 1e-5)
    u = uc[...]
    co = lax.rsqrt(u * u + 1.0)
    out[...] = co * lat[...] + (u * co) * msg


def _mlp(chunk, g, eat, lat, uc, w1c, b1, w2, b2, w3, b3):
    nb = CE // BE
    off = chunk * nb
    blk = lambda r, c: pl.BlockSpec((r, c), lambda i: (i, 0))
    whole = lambda r, c: pl.BlockSpec((r, c), lambda i: (0, 0))
    return pl.pallas_call(
        _mlp_body,
        grid=(nb,),
        in_specs=[
            blk(BE, D),
            pl.BlockSpec((DE, BE), lambda i: (0, off + i)),
            pl.BlockSpec((BE, D), lambda i: (off + i, 0)),
            pl.BlockSpec((BE, 1), lambda i: (off + i, 0)),
            whole(DE, H), whole(1, H),
            whole(H, H), whole(1, H), whole(H, D), whole(1, D),
        ],
        out_specs=blk(BE, D),
        out_shape=jax.ShapeDtypeStruct((CE, D), jnp.float32),
    )(g, eat, lat, uc, w1c, b1, w2, b2, w3, b3)


@functools.cache
def _scatter_kernel(chunk):
    @functools.partial(
        pl.kernel,
        out_type=jax.ShapeDtypeStruct((NC * N, D), jnp.float32),
        mesh=_sc_mesh(),
        scratch_types=[
            pltpu.VMEM((RS,), jnp.int32),
            pltpu.VMEM((RS, D), jnp.float32),
            pltpu.VMEM_SHARED((N, D), jnp.float32),
            pltpu.SemaphoreType.DMA,
        ],
    )
    def _scatter(lat_hbm, dst_hbm, out_hbm, idx_v, buf_v, acc_sh, sem):
        cid = lax.axis_index("c")
        sid = lax.axis_index("s")
        wid = sid * NC + cid

        zero = jnp.zeros((LANES,), jnp.float32)

        def zrow(j, c2):
            for k in range(D // LANES):
                buf_v[j, pl.ds(k * LANES, LANES)] = zero
            return c2

        lax.fori_loop(0, ZR, zrow, 0)
        for k in range(640 // ZR):
            @pl.when((sid < 15) | (k < 400 // ZR))
            def _():
                pltpu.sync_copy(buf_v.at[pl.ds(0, ZR)],
                                acc_sh.at[pl.ds(sid * 640 + k * ZR, ZR)])

        plsc.subcore_barrier()

        def body(i, carry):
            off = wid * EPW + i * RS
            pltpu.sync_copy(dst_hbm.at[pl.ds(chunk * CE + off, RS)], idx_v)
            pltpu.sync_copy(lat_hbm.at[pl.ds(off, RS)], buf_v)
            pltpu.sync_copy(buf_v, acc_sh.at[idx_v], add=True)
            return carry

        lax.fori_loop(0, EPW // RS, body, 0)
        plsc.subcore_barrier()

        @pl.when(sid < 15)
        def _():
            pltpu.sync_copy(acc_sh.at[pl.ds(sid * 640, 640)],
                            out_hbm.at[pl.ds(cid * N + sid * 640, 640)])

        @pl.when(sid == 15)
        def _():
            pltpu.sync_copy(acc_sh.at[pl.ds(9600, 400)],
                            out_hbm.at[pl.ds(cid * N + 9600, 400)])

    return _scatter


def _add_body(*refs):
    out = refs[-1]
    acc = refs[0][...]
    for r in refs[1:-1]:
        acc = acc + r[...]
    out[...] = acc


def _final_add(parts):
    BN = 1000
    return pl.pallas_call(
        _add_body,
        grid=(N // BN,),
        in_specs=[pl.BlockSpec((BN, D), lambda i: (i, 0)) for _ in parts],
        out_specs=pl.BlockSpec((BN, D), lambda i: (i, 0)),
        out_shape=jax.ShapeDtypeStruct((N, D), jnp.float32),
    )(*parts)


def kernel(x, edge_index, edge_attr, latents, update_coeff, W1, b1, W2, b2, W3, b3):
    bf = jnp.bfloat16
    src = edge_index[0]
    dst = edge_index[1]
    w1a = W1[:D].astype(bf)
    w1b = W1[D:2 * D].astype(bf)
    w1c = W1[2 * D:]
    eat = edge_attr.T
    w2b = W2.astype(bf)
    w3b = W3.astype(bf)
    b1r = b1.reshape(1, H)
    b2r = b2.reshape(1, H)
    b3r = b3.reshape(1, D)
    p, q = _pq(x, w1a, w1b)

    parts = []
    for c in range(K):
        g = _gather_kernel(c)(p, q, dst, src)
        new_lat = _mlp(c, g, eat, latents, update_coeff,
                       w1c, b1r, w2b, b2r, w3b, b3r)
        pc = _scatter_kernel(c)(new_lat, dst)
        parts.append(pc[:N])
        parts.append(pc[N:])
    return _final_add(parts)
